# baseline (speedup 1.0000x reference)
import functools

import jax
import jax.numpy as jnp
from jax import lax
from jax.experimental import pallas as pl
from jax.experimental.pallas import tpu as pltpu
from jax.experimental.pallas import tpu_sc as plsc

_info = plsc.get_sparse_core_info()
_NC, _NS, _L = _info.num_cores, _info.num_subcores, _info.num_lanes
_NW = _NC * _NS
_C = 128


@functools.partial(jax.jit, static_argnames=("n", "d"))
def _gather(table, idx, n, d):
    n_chunks = -(-n // (_NW * _C))
    per_w = n_chunks * _C
    full_w = n // per_w
    rem = n - full_w * per_w
    fc, tr = rem // _C, rem % _C
    mesh = plsc.VectorSubcoreMesh(core_axis_name="c", subcore_axis_name="s")

    @functools.partial(
        pl.kernel,
        mesh=mesh,
        out_type=jax.ShapeDtypeStruct((n, d), jnp.float32),
        compiler_params=pltpu.CompilerParams(needs_layout_passes=False),
        scratch_types=[
            pltpu.VMEM((table.shape[0],), jnp.float32),
            pltpu.VMEM((per_w,), jnp.int32),
            pltpu.VMEM((_C, d), jnp.float32),
            pltpu.VMEM((_C, d), jnp.float32),
            pltpu.SemaphoreType.DMA,
            pltpu.SemaphoreType.DMA,
        ],
    )
    def k(table_hbm, idx_hbm, out_hbm, table_v, idx_v, buf0, buf1, ws0, ws1):
        sid = lax.axis_index("s")
        wid = sid * _NC + lax.axis_index("c")
        base = wid * per_w

        pltpu.sync_copy(table_hbm, table_v)

        lanes = lax.iota(jnp.int32, _L)

        def compute_blocks(j, buf, nb, tail_mask_n=0):
            def block(b, carry):
                rows16 = idx_v[pl.ds(j * _C + b * _L, _L)] * d
                dst16 = lanes + b * _L
                for c in range(d):
                    cvec = jnp.full((_L,), c, jnp.int32)
                    x = plsc.load_gather(table_v, [rows16 + c])
                    plsc.store_scatter(buf, [dst16, cvec], x)
                return carry
            lax.fori_loop(0, nb, block, 0)
            if tail_mask_n:
                m = lanes < tail_mask_n
                rows16 = plsc.load_expanded(idx_v.at[pl.ds(j * _C + nb * _L, _L)], m) * d
                dst16 = lanes + nb * _L
                for c in range(d):
                    cvec = jnp.full((_L,), c, jnp.int32)
                    x = plsc.load_gather(table_v, [rows16 + c], mask=m)
                    plsc.store_scatter(buf, [dst16, cvec], x, mask=m)

        def issue_write(j, buf, sem, rows=_C):
            return pltpu.async_copy(
                buf.at[pl.ds(0, rows)] if rows != _C else buf,
                out_hbm.at[pl.ds(base + j * _C, rows)],
                sem,
            )

        def wait_write(j, buf, sem, rows=_C):
            pltpu.make_async_copy(
                buf.at[pl.ds(0, rows)] if rows != _C else buf,
                out_hbm.at[pl.ds(base + j * _C, rows)],
                sem,
            ).wait()

        nblk = _C // _L

        @pl.when(wid < full_w)
        def _full_slab():
            pltpu.sync_copy(idx_hbm.at[pl.ds(base, per_w)], idx_v)
            compute_blocks(0, buf0, nblk)
            issue_write(0, buf0, ws0)
            if n_chunks > 1:
                compute_blocks(1, buf1, nblk)
                issue_write(1, buf1, ws1)
            pairs = (n_chunks - 2) // 2 if n_chunks > 2 else 0

            def body(i, carry):
                j0 = 2 + 2 * i
                wait_write(j0 - 2, buf0, ws0)
                compute_blocks(j0, buf0, nblk)
                issue_write(j0, buf0, ws0)
                wait_write(j0 - 1, buf1, ws1)
                compute_blocks(j0 + 1, buf1, nblk)
                issue_write(j0 + 1, buf1, ws1)
                return carry

            lax.fori_loop(0, pairs, body, 0)
            last = 2 + 2 * pairs
            for j in range(last, n_chunks):
                b, s = (buf0, ws0) if j % 2 == 0 else (buf1, ws1)
                wait_write(j - 2, b, s)
                compute_blocks(j, b, nblk)
                issue_write(j, b, s)
            for j in range(max(0, n_chunks - 2), n_chunks):
                b, s = (buf0, ws0) if j % 2 == 0 else (buf1, ws1)
                wait_write(j, b, s)

        if rem > 0:
            tb, tm = tr // _L, tr % _L

            @pl.when(wid == full_w)
            def _partial_slab():
                pltpu.sync_copy(
                    idx_hbm.at[pl.ds(base, rem)], idx_v.at[pl.ds(0, rem)]
                )
                writes = {}
                for j in range(fc):
                    b, s = (buf0, ws0) if j % 2 == 0 else (buf1, ws1)
                    if j >= 2:
                        wait_write(j - 2, b, s)
                    compute_blocks(j, b, nblk)
                    issue_write(j, b, s)
                    writes[j % 2] = j
                if tr > 0:
                    b, s = (buf0, ws0) if fc % 2 == 0 else (buf1, ws1)
                    if fc >= 2:
                        wait_write(fc - 2, b, s)
                        writes.pop(fc % 2, None)
                    compute_blocks(fc, b, tb, tail_mask_n=tm)
                    issue_write(fc, b, s, rows=tr)
                    wait_write(fc, b, s, rows=tr)
                for p, j in writes.items():
                    b, s = (buf0, ws0) if p == 0 else (buf1, ws1)
                    wait_write(j, b, s)

    return k(table, idx)


def kernel(atomic_numbers, embedding_weight):
    n = atomic_numbers.shape[0]
    d = embedding_weight.shape[1]
    idx = atomic_numbers.astype(jnp.int32)
    return _gather(embedding_weight.reshape(-1), idx, n, d)


# speedup vs baseline: 11.9959x; 11.9959x over previous
import functools

import jax
import jax.numpy as jnp
from jax import lax
from jax.experimental import pallas as pl
from jax.experimental.pallas import tpu as pltpu
from jax.experimental.pallas import tpu_sc as plsc

_info = plsc.get_sparse_core_info()
_NC, _NS = _info.num_cores, _info.num_subcores
_NW = _NC * _NS
_C = 128
_NBUF = 6
_AHEAD = 3


@functools.partial(jax.jit, static_argnames=("n",))
def _gather(table, idx, n):
    d = table.shape[1]
    n_chunks = -(-n // (_NW * _C))
    per_w = n_chunks * _C
    full_w = n // per_w
    rem = n - full_w * per_w
    fc, tr = rem // _C, rem % _C
    mesh = plsc.VectorSubcoreMesh(core_axis_name="c", subcore_axis_name="s")

    @functools.partial(
        pl.kernel,
        mesh=mesh,
        out_type=jax.ShapeDtypeStruct((n, d), jnp.float32),
        scratch_types=[
            pltpu.VMEM_SHARED(table.shape, jnp.float32),
            pltpu.VMEM((per_w,), jnp.int32),
            pltpu.VMEM((_C, d), jnp.float32),
            pltpu.VMEM((_C, d), jnp.float32),
            pltpu.VMEM((_C, d), jnp.float32),
            pltpu.VMEM((_C, d), jnp.float32),
            pltpu.VMEM((_C, d), jnp.float32),
            pltpu.VMEM((_C, d), jnp.float32),
            pltpu.SemaphoreType.DMA,
            pltpu.SemaphoreType.DMA,
            pltpu.SemaphoreType.DMA,
            pltpu.SemaphoreType.DMA,
            pltpu.SemaphoreType.DMA,
            pltpu.SemaphoreType.DMA,
            pltpu.SemaphoreType.DMA,
            pltpu.SemaphoreType.DMA,
            pltpu.SemaphoreType.DMA,
            pltpu.SemaphoreType.DMA,
            pltpu.SemaphoreType.DMA,
            pltpu.SemaphoreType.DMA,
        ],
    )
    def k(table_hbm, idx_hbm, out_hbm, table_v, idx_v,
          buf0, buf1, buf2, buf3, buf4, buf5,
          gs0, gs1, gs2, gs3, gs4, gs5, ws0, ws1, ws2, ws3, ws4, ws5):
        sid = lax.axis_index("s")
        wid = sid * _NC + lax.axis_index("c")
        base = wid * per_w

        @pl.when(sid == 0)
        def _copy_table():
            pltpu.sync_copy(table_hbm, table_v)

        plsc.subcore_barrier()

        bufs = (buf0, buf1, buf2, buf3, buf4, buf5)
        gsems = (gs0, gs1, gs2, gs3, gs4, gs5)
        wsems = (ws0, ws1, ws2, ws3, ws4, ws5)

        def gather_chunk(j, b):
            return pltpu.async_copy(
                table_v.at[idx_v.at[pl.ds(j * _C, _C)]], bufs[b], gsems[b]
            )

        @pl.when(wid < full_w)
        def _full_slab():
            pltpu.sync_copy(idx_hbm.at[pl.ds(base, per_w)], idx_v)
            gathers = [None] * n_chunks
            writes = [None] * _NBUF
            for m in range(min(_AHEAD + 1, n_chunks)):
                gathers[m] = gather_chunk(m, m % _NBUF)
            for j in range(n_chunks):
                gathers[j].wait()
                w = pltpu.async_copy(
                    bufs[j % _NBUF],
                    out_hbm.at[pl.ds(base + j * _C, _C)],
                    wsems[j % _NBUF],
                )
                nxt = j + _AHEAD + 1
                if nxt < n_chunks:
                    b = nxt % _NBUF
                    if writes[b] is not None:
                        writes[b].wait()
                    gathers[nxt] = gather_chunk(nxt, b)
                writes[j % _NBUF] = w
            for b in range(_NBUF):
                if writes[b] is not None:
                    writes[b].wait()

        if rem > 0:
            @pl.when(wid == full_w)
            def _partial_slab():
                pltpu.sync_copy(
                    idx_hbm.at[pl.ds(base, rem)], idx_v.at[pl.ds(0, rem)]
                )
                for j in range(fc):
                    gather_chunk(j, j % _NBUF).wait()
                    pltpu.sync_copy(
                        bufs[j % _NBUF], out_hbm.at[pl.ds(base + j * _C, _C)]
                    )
                if tr > 0:
                    pltpu.async_copy(
                        table_v.at[idx_v.at[pl.ds(fc * _C, tr)]],
                        bufs[fc % _NBUF].at[pl.ds(0, tr)],
                        gsems[fc % _NBUF],
                    ).wait()
                    pltpu.sync_copy(
                        bufs[fc % _NBUF].at[pl.ds(0, tr)],
                        out_hbm.at[pl.ds(base + fc * _C, tr)],
                    )

    return k(table, idx)


def kernel(atomic_numbers, embedding_weight):
    n = atomic_numbers.shape[0]
    idx = atomic_numbers.astype(jnp.int32)
    return _gather(embedding_weight, idx, n)


# speedup vs baseline: 12.0326x; 1.0031x over previous
import functools

import jax
import jax.numpy as jnp
from jax import lax
from jax.experimental import pallas as pl
from jax.experimental.pallas import tpu as pltpu
from jax.experimental.pallas import tpu_sc as plsc

_info = plsc.get_sparse_core_info()
_NC, _NS = _info.num_cores, _info.num_subcores
_NW = _NC * _NS
_C = 128
_NBUF = 4
_AHEAD = 2


@functools.partial(jax.jit, static_argnames=("n",))
def _gather(table, idx, n):
    d = table.shape[1]
    n_chunks = -(-n // (_NW * _C))
    per_w = n_chunks * _C
    full_w = n // per_w
    rem = n - full_w * per_w
    fc, tr = rem // _C, rem % _C
    mesh = plsc.VectorSubcoreMesh(core_axis_name="c", subcore_axis_name="s")

    @functools.partial(
        pl.kernel,
        mesh=mesh,
        out_type=jax.ShapeDtypeStruct((n, d), jnp.float32),
        scratch_types=[
            pltpu.VMEM_SHARED(table.shape, jnp.float32),
            pltpu.VMEM((per_w,), jnp.int32),
            pltpu.VMEM((_C, d), jnp.float32),
            pltpu.VMEM((_C, d), jnp.float32),
            pltpu.VMEM((_C, d), jnp.float32),
            pltpu.VMEM((_C, d), jnp.float32),
            pltpu.SemaphoreType.DMA,
            pltpu.SemaphoreType.DMA,
            pltpu.SemaphoreType.DMA,
            pltpu.SemaphoreType.DMA,
            pltpu.SemaphoreType.DMA,
            pltpu.SemaphoreType.DMA,
            pltpu.SemaphoreType.DMA,
            pltpu.SemaphoreType.DMA,
        ],
    )
    def k(table_hbm, idx_hbm, out_hbm, table_v, idx_v,
          buf0, buf1, buf2, buf3, gs0, gs1, gs2, gs3, ws0, ws1, ws2, ws3):
        sid = lax.axis_index("s")
        wid = sid * _NC + lax.axis_index("c")
        base = wid * per_w

        @pl.when(sid == 0)
        def _copy_table():
            pltpu.sync_copy(table_hbm, table_v)

        plsc.subcore_barrier()

        bufs = (buf0, buf1, buf2, buf3)
        gsems = (gs0, gs1, gs2, gs3)
        wsems = (ws0, ws1, ws2, ws3)

        def gather_chunk(j, b):
            return pltpu.async_copy(
                table_v.at[idx_v.at[pl.ds(j * _C, _C)]], bufs[b], gsems[b]
            )

        @pl.when(wid < full_w)
        def _full_slab():
            pltpu.sync_copy(idx_hbm.at[pl.ds(base, per_w)], idx_v)
            gathers = [None] * n_chunks
            writes = [None] * _NBUF
            for m in range(min(_AHEAD + 1, n_chunks)):
                gathers[m] = gather_chunk(m, m % _NBUF)
            for j in range(n_chunks):
                gathers[j].wait()
                w = pltpu.async_copy(
                    bufs[j % _NBUF],
                    out_hbm.at[pl.ds(base + j * _C, _C)],
                    wsems[j % _NBUF],
                )
                nxt = j + _AHEAD + 1
                if nxt < n_chunks:
                    b = nxt % _NBUF
                    if writes[b] is not None:
                        writes[b].wait()
                    gathers[nxt] = gather_chunk(nxt, b)
                writes[j % _NBUF] = w
            for b in range(_NBUF):
                if writes[b] is not None:
                    writes[b].wait()

        if rem > 0:
            @pl.when(wid == full_w)
            def _partial_slab():
                pltpu.sync_copy(
                    idx_hbm.at[pl.ds(base, rem)], idx_v.at[pl.ds(0, rem)]
                )
                for j in range(fc):
                    gather_chunk(j, j % _NBUF).wait()
                    pltpu.sync_copy(
                        bufs[j % _NBUF], out_hbm.at[pl.ds(base + j * _C, _C)]
                    )
                if tr > 0:
                    pltpu.async_copy(
                        table_v.at[idx_v.at[pl.ds(fc * _C, tr)]],
                        bufs[fc % _NBUF].at[pl.ds(0, tr)],
                        gsems[fc % _NBUF],
                    ).wait()
                    pltpu.sync_copy(
                        bufs[fc % _NBUF].at[pl.ds(0, tr)],
                        out_hbm.at[pl.ds(base + fc * _C, tr)],
                    )

    return k(table, idx)


def kernel(atomic_numbers, embedding_weight):
    n = atomic_numbers.shape[0]
    idx = atomic_numbers.astype(jnp.int32)
    return _gather(embedding_weight, idx, n)


# speedup vs baseline: 12.0597x; 1.0022x over previous
import functools

import jax
import jax.numpy as jnp
from jax import lax
from jax.experimental import pallas as pl
from jax.experimental.pallas import tpu as pltpu
from jax.experimental.pallas import tpu_sc as plsc

_info = plsc.get_sparse_core_info()
_NC, _NS = _info.num_cores, _info.num_subcores
_NW = _NC * _NS
_C = 128
_NBUF = 4
_AHEAD = 2


@functools.partial(jax.jit, static_argnames=("n",))
def _gather(table, idx, n):
    d = table.shape[1]
    n_chunks = -(-n // (_NW * _C))
    per_w = n_chunks * _C
    full_w = n // per_w
    rem = n - full_w * per_w
    fc, tr = rem // _C, rem % _C
    mesh = plsc.VectorSubcoreMesh(core_axis_name="c", subcore_axis_name="s")

    @functools.partial(
        pl.kernel,
        mesh=mesh,
        out_type=jax.ShapeDtypeStruct((n, d), jnp.float32),
        scratch_types=[
            pltpu.VMEM_SHARED(table.shape, jnp.float32),
            pltpu.VMEM((per_w,), jnp.int32),
            pltpu.VMEM((_C, d), jnp.float32),
            pltpu.VMEM((_C, d), jnp.float32),
            pltpu.VMEM((_C, d), jnp.float32),
            pltpu.VMEM((_C, d), jnp.float32),
            pltpu.SemaphoreType.DMA,
            pltpu.SemaphoreType.DMA,
            pltpu.SemaphoreType.DMA,
            pltpu.SemaphoreType.DMA,
            pltpu.SemaphoreType.DMA,
            pltpu.SemaphoreType.DMA,
            pltpu.SemaphoreType.DMA,
            pltpu.SemaphoreType.DMA,
        ],
    )
    def k(table_hbm, idx_hbm, out_hbm, table_v, idx_v,
          buf0, buf1, buf2, buf3, gs0, gs1, gs2, gs3, ws0, ws1, ws2, ws3):
        sid = lax.axis_index("s")
        wid = sid * _NC + lax.axis_index("c")
        base = wid * per_w

        @pl.when(sid == 0)
        def _copy_table():
            pltpu.sync_copy(table_hbm, table_v)

        @pl.when(wid < full_w)
        def _copy_idx_full():
            pltpu.sync_copy(idx_hbm.at[pl.ds(base, per_w)], idx_v)

        if rem > 0:
            @pl.when(wid == full_w)
            def _copy_idx_partial():
                pltpu.sync_copy(
                    idx_hbm.at[pl.ds(base, rem)], idx_v.at[pl.ds(0, rem)]
                )

        plsc.subcore_barrier()

        bufs = (buf0, buf1, buf2, buf3)
        gsems = (gs0, gs1, gs2, gs3)
        wsems = (ws0, ws1, ws2, ws3)

        def gather_chunk(j, b):
            return pltpu.async_copy(
                table_v.at[idx_v.at[pl.ds(j * _C, _C)]], bufs[b], gsems[b]
            )

        @pl.when(wid < full_w)
        def _full_slab():
            gathers = [None] * n_chunks
            writes = [None] * _NBUF
            for m in range(min(_AHEAD + 1, n_chunks)):
                gathers[m] = gather_chunk(m, m % _NBUF)
            for j in range(n_chunks):
                gathers[j].wait()
                w = pltpu.async_copy(
                    bufs[j % _NBUF],
                    out_hbm.at[pl.ds(base + j * _C, _C)],
                    wsems[j % _NBUF],
                )
                nxt = j + _AHEAD + 1
                if nxt < n_chunks:
                    b = nxt % _NBUF
                    if writes[b] is not None:
                        writes[b].wait()
                    gathers[nxt] = gather_chunk(nxt, b)
                writes[j % _NBUF] = w
            for b in range(_NBUF):
                if writes[b] is not None:
                    writes[b].wait()

        if rem > 0:
            @pl.when(wid == full_w)
            def _partial_slab():
                for j in range(fc):
                    gather_chunk(j, j % _NBUF).wait()
                    pltpu.sync_copy(
                        bufs[j % _NBUF], out_hbm.at[pl.ds(base + j * _C, _C)]
                    )
                if tr > 0:
                    pltpu.async_copy(
                        table_v.at[idx_v.at[pl.ds(fc * _C, tr)]],
                        bufs[fc % _NBUF].at[pl.ds(0, tr)],
                        gsems[fc % _NBUF],
                    ).wait()
                    pltpu.sync_copy(
                        bufs[fc % _NBUF].at[pl.ds(0, tr)],
                        out_hbm.at[pl.ds(base + fc * _C, tr)],
                    )

    return k(table, idx)


def kernel(atomic_numbers, embedding_weight):
    n = atomic_numbers.shape[0]
    idx = atomic_numbers.astype(jnp.int32)
    return _gather(embedding_weight, idx, n)


# speedup vs baseline: 12.0623x; 1.0002x over previous
import functools

import jax
import jax.numpy as jnp
from jax import lax
from jax.experimental import pallas as pl
from jax.experimental.pallas import tpu as pltpu
from jax.experimental.pallas import tpu_sc as plsc

_info = plsc.get_sparse_core_info()
_NC, _NS = _info.num_cores, _info.num_subcores
_NW = _NC * _NS
_C = 128
_NBUF = 4
_AHEAD = 2


@functools.partial(jax.jit, static_argnames=("n",))
def _gather(table, idx, n):
    d = table.shape[1]
    n_chunks = -(-n // (_NW * _C))
    per_w = n_chunks * _C
    full_w = n // per_w
    rem = n - full_w * per_w
    fc, tr = rem // _C, rem % _C
    mesh = plsc.VectorSubcoreMesh(core_axis_name="c", subcore_axis_name="s")

    @functools.partial(
        pl.kernel,
        mesh=mesh,
        out_type=jax.ShapeDtypeStruct((n, d), jnp.float32),
        compiler_params=pltpu.CompilerParams(skip_device_barrier=True),
        scratch_types=[
            pltpu.VMEM_SHARED(table.shape, jnp.float32),
            pltpu.VMEM((per_w,), jnp.int32),
            pltpu.VMEM((_C, d), jnp.float32),
            pltpu.VMEM((_C, d), jnp.float32),
            pltpu.VMEM((_C, d), jnp.float32),
            pltpu.VMEM((_C, d), jnp.float32),
            pltpu.SemaphoreType.DMA,
            pltpu.SemaphoreType.DMA,
            pltpu.SemaphoreType.DMA,
            pltpu.SemaphoreType.DMA,
            pltpu.SemaphoreType.DMA,
            pltpu.SemaphoreType.DMA,
            pltpu.SemaphoreType.DMA,
            pltpu.SemaphoreType.DMA,
        ],
    )
    def k(table_hbm, idx_hbm, out_hbm, table_v, idx_v,
          buf0, buf1, buf2, buf3, gs0, gs1, gs2, gs3, ws0, ws1, ws2, ws3):
        sid = lax.axis_index("s")
        wid = sid * _NC + lax.axis_index("c")
        base = wid * per_w

        @pl.when(sid == 0)
        def _copy_table():
            pltpu.sync_copy(table_hbm, table_v)

        @pl.when(wid < full_w)
        def _copy_idx_full():
            pltpu.sync_copy(idx_hbm.at[pl.ds(base, per_w)], idx_v)

        if rem > 0:
            @pl.when(wid == full_w)
            def _copy_idx_partial():
                pltpu.sync_copy(
                    idx_hbm.at[pl.ds(base, rem)], idx_v.at[pl.ds(0, rem)]
                )

        plsc.subcore_barrier()

        bufs = (buf0, buf1, buf2, buf3)
        gsems = (gs0, gs1, gs2, gs3)
        wsems = (ws0, ws1, ws2, ws3)

        def gather_chunk(j, b):
            return pltpu.async_copy(
                table_v.at[idx_v.at[pl.ds(j * _C, _C)]], bufs[b], gsems[b]
            )

        @pl.when(wid < full_w)
        def _full_slab():
            gathers = [None] * n_chunks
            writes = [None] * _NBUF
            for m in range(min(_AHEAD + 1, n_chunks)):
                gathers[m] = gather_chunk(m, m % _NBUF)
            for j in range(n_chunks):
                gathers[j].wait()
                w = pltpu.async_copy(
                    bufs[j % _NBUF],
                    out_hbm.at[pl.ds(base + j * _C, _C)],
                    wsems[j % _NBUF],
                )
                nxt = j + _AHEAD + 1
                if nxt < n_chunks:
                    b = nxt % _NBUF
                    if writes[b] is not None:
                        writes[b].wait()
                    gathers[nxt] = gather_chunk(nxt, b)
                writes[j % _NBUF] = w
            for b in range(_NBUF):
                if writes[b] is not None:
                    writes[b].wait()

        if rem > 0:
            @pl.when(wid == full_w)
            def _partial_slab():
                for j in range(fc):
                    gather_chunk(j, j % _NBUF).wait()
                    pltpu.sync_copy(
                        bufs[j % _NBUF], out_hbm.at[pl.ds(base + j * _C, _C)]
                    )
                if tr > 0:
                    pltpu.async_copy(
                        table_v.at[idx_v.at[pl.ds(fc * _C, tr)]],
                        bufs[fc % _NBUF].at[pl.ds(0, tr)],
                        gsems[fc % _NBUF],
                    ).wait()
                    pltpu.sync_copy(
                        bufs[fc % _NBUF].at[pl.ds(0, tr)],
                        out_hbm.at[pl.ds(base + fc * _C, tr)],
                    )

    return k(table, idx)


def kernel(atomic_numbers, embedding_weight):
    n = atomic_numbers.shape[0]
    idx = atomic_numbers.astype(jnp.int32)
    return _gather(embedding_weight, idx, n)
